# Initial kernel scaffold; baseline (speedup 1.0000x reference)
import jax
import jax.numpy as jnp
from jax.experimental import pallas as pl


def kernel(x, edge_index, W1_l, b1, W1_r, W2_l, b2, W2_r):
    raise NotImplementedError("write your pallas kernel here")


# speedup vs baseline: 5.4572x; 5.4572x over previous
import functools

import jax
import jax.numpy as jnp
from jax import lax
from jax.experimental import pallas as pl
from jax.experimental.pallas import tpu as pltpu
from jax.experimental.pallas import tpu_sc as plsc

N_NODES = 10000
N_EDGES = 320000
IN_FEATS = 128
HIDDEN = 128
NUM_CLASSES = 64

NC = 2
NS = 16
NW = NC * NS
CHUNK = 128
CHUNKS = 79
E_PAD = NW * CHUNKS * CHUNK
ROWS_PER_TILE = 640
N_PAD = NS * ROWS_PER_TILE
INIT_STEPS = ROWS_PER_TILE // CHUNK


def _sc_aggregate(d, with_cnt):
    mesh = plsc.VectorSubcoreMesh(core_axis_name="c", subcore_axis_name="s")
    out_type = [jax.ShapeDtypeStruct((NC, N_PAD, d), jnp.float32)]
    scratch = [
        pltpu.VMEM_SHARED((N_PAD, d), jnp.float32),
        pltpu.VMEM((2, CHUNK), jnp.int32),
        pltpu.VMEM((CHUNK, d), jnp.float32),
        pltpu.SemaphoreType.DMA,
    ]
    if with_cnt:
        out_type.append(jax.ShapeDtypeStruct((NC, N_PAD, 16), jnp.float32))
        scratch += [
            pltpu.VMEM_SHARED((N_PAD, 16), jnp.float32),
            pltpu.VMEM((CHUNK, 16), jnp.float32),
        ]

    def body(y, eR, z, z16, ones16, out, cntout, acc, idx, rows, sem,
             cntacc=None, ones_v=None):
        c = lax.axis_index("c")
        s = lax.axis_index("s")
        w = c * NS + s
        off = s * ROWS_PER_TILE

        pltpu.sync_copy(z, rows)
        for t in range(INIT_STEPS):
            pltpu.sync_copy(rows, acc.at[pl.ds(off + t * CHUNK, CHUNK)])
        if with_cnt:
            pltpu.sync_copy(z16, ones_v)
            for t in range(INIT_STEPS):
                pltpu.sync_copy(ones_v, cntacc.at[pl.ds(off + t * CHUNK, CHUNK)])
            pltpu.sync_copy(ones16, ones_v)
        plsc.subcore_barrier()

        def step(j, carry):
            pltpu.sync_copy(eR.at[w, j], idx)
            pltpu.async_copy(y.at[idx.at[0]], rows, sem).wait()
            pltpu.sync_copy(rows, acc.at[idx.at[1]], add=True)
            if with_cnt:
                pltpu.sync_copy(ones_v, cntacc.at[idx.at[1]], add=True)
            return carry

        lax.fori_loop(0, CHUNKS, step, 0)
        plsc.subcore_barrier()

        for t in range(INIT_STEPS):
            r0 = off + t * CHUNK
            pltpu.sync_copy(acc.at[pl.ds(r0, CHUNK)], rows)
            pltpu.sync_copy(rows, out.at[c, pl.ds(r0, CHUNK)])
            if with_cnt:
                pltpu.sync_copy(cntacc.at[pl.ds(r0, CHUNK)], ones_v)
                pltpu.sync_copy(ones_v, cntout.at[c, pl.ds(r0, CHUNK)])

    if with_cnt:
        def body_cnt(y, eR, z, z16, ones16, out, cntout, acc, idx, rows,
                     sem, cntacc, ones_v):
            body(y, eR, z, z16, ones16, out, cntout, acc, idx, rows, sem,
                 cntacc, ones_v)
        fn = pl.kernel(body_cnt, mesh=mesh, out_type=out_type,
                       scratch_types=scratch,
                       compiler_params=pltpu.CompilerParams(
                           use_tc_tiling_on_sc=False))
        return fn
    else:
        def body_nocnt(y, eR, z, out, acc, idx, rows, sem):
            body(y, eR, z, None, None, out, None, acc, idx, rows, sem)
        fn = pl.kernel(body_nocnt, mesh=mesh, out_type=out_type,
                       scratch_types=scratch,
                       compiler_params=pltpu.CompilerParams(
                           use_tc_tiling_on_sc=False))
        return fn


def _tc_pre_body(x_ref, wl_ref, wr_ref, b_ref, y_ref, r_ref):
    xb = x_ref[...]
    y_ref[...] = jnp.dot(xb, wl_ref[...], preferred_element_type=jnp.float32)
    r_ref[...] = (
        jnp.dot(xb, wr_ref[...], preferred_element_type=jnp.float32)
        + b_ref[...]
    )


def _tc_mid_body(p0_ref, p1_ref, c0_ref, c1_ref, r1_ref, w2l_ref, w2r_ref,
                 b2_ref, y2_ref, r2_ref):
    cnt = jnp.maximum(c0_ref[:, 0:1] + c1_ref[:, 0:1], 1.0)
    h = jnp.maximum((p0_ref[...] + p1_ref[...]) / cnt + r1_ref[...], 0.0)
    y2_ref[...] = jnp.dot(h, w2l_ref[...], preferred_element_type=jnp.float32)
    r2_ref[...] = (
        jnp.dot(h, w2r_ref[...], preferred_element_type=jnp.float32)
        + b2_ref[...]
    )


def _tc_post_body(q0_ref, q1_ref, c0_ref, c1_ref, r2_ref, out_ref):
    cnt = jnp.maximum(c0_ref[:, 0:1] + c1_ref[:, 0:1], 1.0)
    out_ref[...] = (q0_ref[...] + q1_ref[...]) / cnt + r2_ref[...]


def kernel(x, edge_index, W1_l, b1, W1_r, W2_l, b2, W2_r):
    src = edge_index[0].astype(jnp.int32)
    dst = edge_index[1].astype(jnp.int32)
    pad = E_PAD - N_EDGES
    srcR = jnp.concatenate([src, jnp.zeros((pad,), jnp.int32)]).reshape(
        NW, CHUNKS, 1, CHUNK)
    dstR = jnp.concatenate(
        [dst, jnp.full((pad,), N_NODES, jnp.int32)]).reshape(
        NW, CHUNKS, 1, CHUNK)
    eR = jnp.concatenate([srcR, dstR], axis=2)
    z128 = jnp.zeros((CHUNK, HIDDEN), jnp.float32)
    z64 = jnp.zeros((CHUNK, NUM_CLASSES), jnp.float32)
    z16 = jnp.zeros((CHUNK, 16), jnp.float32)
    ones16 = jnp.ones((CHUNK, 16), jnp.float32)

    y1, r1 = pl.pallas_call(
        _tc_pre_body,
        out_shape=[
            jax.ShapeDtypeStruct((N_NODES, HIDDEN), jnp.float32),
            jax.ShapeDtypeStruct((N_NODES, HIDDEN), jnp.float32),
        ],
    )(x, W1_l, W1_r, b1.reshape(1, HIDDEN))

    p, cntp = _sc_aggregate(HIDDEN, True)(y1, eR, z128, z16, ones16)
    p0 = p[0, :N_NODES]
    p1 = p[1, :N_NODES]
    c0 = cntp[0, :N_NODES]
    c1 = cntp[1, :N_NODES]

    y2, r2 = pl.pallas_call(
        _tc_mid_body,
        out_shape=[
            jax.ShapeDtypeStruct((N_NODES, NUM_CLASSES), jnp.float32),
            jax.ShapeDtypeStruct((N_NODES, NUM_CLASSES), jnp.float32),
        ],
    )(p0, p1, c0, c1, r1, W2_l, W2_r, b2.reshape(1, NUM_CLASSES))

    (q,) = _sc_aggregate(NUM_CLASSES, False)(y2, eR, z64)
    q0 = q[0, :N_NODES]
    q1 = q[1, :N_NODES]

    out = pl.pallas_call(
        _tc_post_body,
        out_shape=jax.ShapeDtypeStruct((N_NODES, NUM_CLASSES), jnp.float32),
    )(q0, q1, c0, c1, r2)
    return out
